# baseline (speedup 1.0000x reference)
import jax
import jax.numpy as jnp
from jax.experimental import pallas as pl
from jax.experimental.pallas import tpu as pltpu

N = 4096
D_IN = 128
D_OUT = 128
NUM_BASES = 8
NUM_REL = 4

BLOCK_N = 512


def _rgcn_kernel(a_ref, xf_ref, wb_ref, wr_ref, out_ref):
    wb = wb_ref[...]
    wr = wr_ref[...]
    w = jax.lax.dot_general(
        wr, wb.reshape(NUM_BASES, D_IN * D_OUT),
        (((1,), (0,)), ((), ())),
        preferred_element_type=jnp.float32,
    ).reshape(NUM_REL, D_IN, D_OUT)
    wp = jnp.transpose(w, (1, 0, 2)).reshape(NUM_REL * D_IN, D_OUT)
    y = jnp.dot(xf_ref[...], wp.astype(jnp.bfloat16),
                preferred_element_type=jnp.float32)
    out_ref[...] = jnp.dot(a_ref[...].astype(jnp.bfloat16),
                           y.astype(jnp.bfloat16),
                           preferred_element_type=jnp.float32)


def kernel(a, x, w_bases, w_rel):
    xf = x.astype(jnp.bfloat16).reshape(N, D_IN * NUM_REL)
    return pl.pallas_call(
        _rgcn_kernel,
        grid=(N // BLOCK_N,),
        in_specs=[
            pl.BlockSpec((BLOCK_N, N), lambda i: (i, 0)),
            pl.BlockSpec((N, D_IN * NUM_REL), lambda i: (0, 0)),
            pl.BlockSpec((NUM_BASES, D_IN, D_OUT), lambda i: (0, 0, 0)),
            pl.BlockSpec((NUM_REL, NUM_BASES), lambda i: (0, 0)),
        ],
        out_specs=pl.BlockSpec((BLOCK_N, D_OUT), lambda i: (i, 0)),
        out_shape=jax.ShapeDtypeStruct((N, D_OUT), jnp.float32),
        compiler_params=pltpu.CompilerParams(
            dimension_semantics=("parallel",),
        ),
    )(a, xf, w_bases, w_rel)


# speedup vs baseline: 1.1971x; 1.1971x over previous
import jax
import jax.numpy as jnp
from jax.experimental import pallas as pl
from jax.experimental.pallas import tpu as pltpu

N = 4096
D_IN = 128
D_OUT = 128
NUM_BASES = 8
NUM_REL = 4

BLOCK_N = 512


NUM_CORES = 2


def _rgcn_kernel(a_ref, xf_ref, wb_ref, wr_ref, out_ref, y_ref):
    @pl.when(pl.program_id(1) == 0)
    def _compute_y():
        wb = wb_ref[...]
        wr = wr_ref[...]
        w = jax.lax.dot_general(
            wr, wb.reshape(NUM_BASES, D_IN * D_OUT),
            (((1,), (0,)), ((), ())),
            preferred_element_type=jnp.float32,
        ).reshape(NUM_REL, D_IN, D_OUT)
        wp = jnp.transpose(w, (1, 0, 2)).reshape(NUM_REL * D_IN, D_OUT)
        y = jnp.dot(xf_ref[...], wp.astype(jnp.bfloat16),
                    preferred_element_type=jnp.float32)
        y_ref[...] = y.astype(jnp.bfloat16)

    out_ref[...] = jnp.dot(a_ref[...].astype(jnp.bfloat16), y_ref[...],
                           preferred_element_type=jnp.float32)


def kernel(a, x, w_bases, w_rel):
    xf = x.astype(jnp.bfloat16).reshape(N, D_IN * NUM_REL)
    inner = N // (NUM_CORES * BLOCK_N)
    return pl.pallas_call(
        _rgcn_kernel,
        grid=(NUM_CORES, inner),
        in_specs=[
            pl.BlockSpec((BLOCK_N, N), lambda i, j: (i * inner + j, 0)),
            pl.BlockSpec((N, D_IN * NUM_REL), lambda i, j: (0, 0)),
            pl.BlockSpec((NUM_BASES, D_IN, D_OUT), lambda i, j: (0, 0, 0)),
            pl.BlockSpec((NUM_REL, NUM_BASES), lambda i, j: (0, 0)),
        ],
        out_specs=pl.BlockSpec((BLOCK_N, D_OUT), lambda i, j: (i * inner + j, 0)),
        out_shape=jax.ShapeDtypeStruct((N, D_OUT), jnp.float32),
        scratch_shapes=[pltpu.VMEM((N, D_OUT), jnp.bfloat16)],
        compiler_params=pltpu.CompilerParams(
            dimension_semantics=("parallel", "arbitrary"),
        ),
    )(a, xf, w_bases, w_rel)
